# Initial kernel scaffold; baseline (speedup 1.0000x reference)
import jax
import jax.numpy as jnp
from jax.experimental import pallas as pl


def kernel(token_ids, table, W_out, hash_mults, hash_bias):
    raise NotImplementedError("write your pallas kernel here")


# speedup vs baseline: 3.8231x; 3.8231x over previous
import numpy as np
import jax
import jax.numpy as jnp
from jax import lax
from jax.experimental import pallas as pl
from jax.experimental.pallas import tpu as pltpu
from jax.experimental.pallas import tpu_sc as plsc

NUM_TABLES = 16
EMBED_DIM = 32
B, S = 4, 2048
P_TOTAL = B * S
NW = 32
PPW = P_TOTAL // NW
CHUNK_P = 8
N_CHUNKS = PPW // CHUNK_P

_SIZES = np.array([100000 + i for i in range(NUM_TABLES)], dtype=np.int64)
_OFFS = np.concatenate([[0], np.cumsum(_SIZES)[:-1]]).astype(np.int64)
_C32 = np.array([(1 << 32) % int(s) for s in _SIZES], dtype=np.int64)
TOTAL_ROWS = int(_SIZES.sum())
N_EMBD = 1024


def _u32(x):
    return plsc.bitcast(x, jnp.uint32)


def _sc_body(table_hbm, tok_hbm, consts_hbm, emb_hbm,
             tok_v, consts_v, idx_v, row_v, gsem):
    cid = lax.axis_index("c")
    sid = lax.axis_index("s")
    wid = sid * 2 + cid
    base = wid * PPW

    pltpu.sync_copy(tok_hbm.at[pl.ds(wid * (3 * PPW), 3 * PPW)], tok_v)
    pltpu.sync_copy(consts_hbm, consts_v)

    ml0 = _u32(consts_v[0, :])
    mh0 = _u32(consts_v[1, :])
    ml1 = _u32(consts_v[2, :])
    mh1 = _u32(consts_v[3, :])
    ml2 = _u32(consts_v[4, :])
    mh2 = _u32(consts_v[5, :])
    bias = _u32(consts_v[6, :])
    sizes = _u32(consts_v[7, :])
    c32 = _u32(consts_v[8, :])
    offs = _u32(consts_v[9, :])
    lane = lax.iota(jnp.int32, 16)
    is_tri = lane >= 8
    zero = jnp.zeros((16,), jnp.uint32)
    one = jnp.full((16,), 1, jnp.uint32)
    s16 = jnp.full((16,), 16, jnp.uint32)

    def _prod(ml, mh, tb):
        a = ml * tb
        bb = mh * tb
        lo = a + (bb << s16)
        carry = jnp.where(lo < a, one, zero)
        hi = (bb >> s16) + carry
        return lo, hi

    dnums = lax.GatherDimensionNumbers(
        offset_dims=(), collapsed_slice_dims=(0,), start_index_map=(0,))

    def _bcast(vec, j):
        jidx = jnp.full((16, 1), j, dtype=jnp.int32)
        return lax.gather(vec, jidx, dnums, (1,),
                          mode=lax.GatherScatterMode.PROMISE_IN_BOUNDS)

    def group_step(_, g):
        goff = g * 16
        t0g = tok_v[pl.ds(goff, 16)]
        t1g = tok_v[pl.ds(goff + PPW, 16)]
        t2g = tok_v[pl.ds(goff + 2 * PPW, 16)]
        for j in range(16):
            t0 = _u32(_bcast(t0g, j))
            t1 = _u32(_bcast(t1g, j))
            t2 = jnp.where(is_tri, _u32(_bcast(t2g, j)), zero)
            lo0, hi0 = _prod(ml0, mh0, t0)
            lo1, hi1 = _prod(ml1, mh1, t1)
            lo2, hi2 = _prod(ml2, mh2, t2)
            h_lo = lo0 ^ lo1 ^ lo2 ^ bias
            h_hi = hi0 ^ hi1 ^ hi2
            r = lax.rem(h_lo, sizes)
            acc = h_hi * c32 + r
            r2 = lax.rem(acc, sizes)
            idx = plsc.bitcast(r2 + offs, jnp.int32)
            row = g * 2 + (j >> 3)
            col = (j & 7) * NUM_TABLES
            idx_v[row, pl.ds(col, NUM_TABLES)] = idx
        return g + 1

    lax.fori_loop(0, PPW // 16, group_step, np.int32(0))

    i32 = jnp.int32
    descs = [None, None]
    descs[0] = pltpu.make_async_copy(
        table_hbm.at[idx_v.at[i32(0)]], row_v.at[i32(0)], gsem[0])
    descs[0].start()
    for c in range(N_CHUNKS):
        buf = c % 2
        if c + 1 < N_CHUNKS:
            nbuf = (c + 1) % 2
            descs[nbuf] = pltpu.make_async_copy(
                table_hbm.at[idx_v.at[i32(c + 1)]], row_v.at[i32(nbuf)],
                gsem[nbuf])
            descs[nbuf].start()
        descs[buf].wait()
        pltpu.sync_copy(
            row_v.at[i32(buf)],
            emb_hbm.at[pl.ds(base * NUM_TABLES + c * CHUNK_P * NUM_TABLES,
                             CHUNK_P * NUM_TABLES)])


def _sc_gather(table, tok_arr, consts):
    mesh = plsc.VectorSubcoreMesh(core_axis_name="c", subcore_axis_name="s")
    fn = pl.kernel(
        _sc_body,
        out_type=jax.ShapeDtypeStruct((P_TOTAL * NUM_TABLES, EMBED_DIM),
                                      jnp.float32),
        mesh=mesh,
        scratch_types=[
            pltpu.VMEM((3 * PPW,), jnp.int32),
            pltpu.VMEM((10, 16), jnp.int32),
            pltpu.VMEM((N_CHUNKS, CHUNK_P * NUM_TABLES), jnp.int32),
            pltpu.VMEM((2, CHUNK_P * NUM_TABLES, EMBED_DIM), jnp.float32),
            [pltpu.SemaphoreType.DMA, pltpu.SemaphoreType.DMA],
        ],
        compiler_params=pltpu.CompilerParams(use_tc_tiling_on_sc=False),
    )
    return fn(table, tok_arr, consts)


def _mm_body(x_ref, w_ref, o_ref):
    o_ref[...] = lax.dot_general(
        x_ref[...], w_ref[...], (((1,), (1,)), ((), ())),
        preferred_element_type=jnp.float32)


def _matmul(x, w):
    m_blk = 1024
    grid = (x.shape[0] // m_blk,)
    return pl.pallas_call(
        _mm_body,
        grid=grid,
        in_specs=[
            pl.BlockSpec((m_blk, NUM_TABLES * EMBED_DIM),
                         lambda i: (i, jnp.int32(0))),
            pl.BlockSpec((N_EMBD, NUM_TABLES * EMBED_DIM),
                         lambda i: (jnp.int32(0), jnp.int32(0))),
        ],
        out_specs=pl.BlockSpec((m_blk, N_EMBD), lambda i: (i, jnp.int32(0))),
        out_shape=jax.ShapeDtypeStruct((x.shape[0], N_EMBD), jnp.float32),
    )(x, w)


def kernel(token_ids, table, W_out, hash_mults, hash_bias):
    tok32 = token_ids.astype(jnp.int32)
    sh0 = tok32
    sh1 = jnp.pad(tok32[:, :-1], ((0, 0), (1, 0)))
    sh2 = jnp.pad(tok32[:, :-2], ((0, 0), (2, 0)))
    stk = jnp.stack([sh0, sh1, sh2], axis=0)
    tok_arr = stk.reshape(3, NW, PPW).transpose(1, 0, 2).reshape(-1)

    mt = hash_mults.T
    ml = (mt & 0xFFFF).astype(jnp.int32)
    mh = (mt >> 16).astype(jnp.int32)
    consts = jnp.stack([
        ml[0], mh[0], ml[1], mh[1], ml[2], mh[2],
        hash_bias.astype(jnp.int32),
        jnp.asarray(_SIZES, jnp.int32),
        jnp.asarray(_C32, jnp.int32),
        jnp.asarray(_OFFS, jnp.int32),
    ])

    emb = _sc_gather(table, tok_arr, consts)
    x = emb.reshape(P_TOTAL, NUM_TABLES * EMBED_DIM)
    out = _matmul(x, W_out)
    return out.reshape(B, S, N_EMBD)
